# baseline (speedup 1.0000x reference)
import functools

import jax
import jax.numpy as jnp
from jax.experimental import pallas as pl
from jax.experimental.pallas import tpu as pltpu
from jax.experimental.pallas import tpu_sc as plsc

_GATHER_WINDOW = 1024


def _sc_gather(sino_t, idx_flat):
    n = idx_flat.shape[1]
    b = sino_t.shape[1]
    mesh = plsc.VectorSubcoreMesh(core_axis_name="core",
                                  subcore_axis_name="subcore")

    @functools.partial(
        pl.kernel,
        out_type=jax.ShapeDtypeStruct((n, b), sino_t.dtype),
        mesh=mesh,
        compiler_params=pltpu.CompilerParams(use_tc_tiling_on_sc=False),
    )
    def gather_kernel(x_hbm, i_hbm, o_hbm):
        def body(i_vmem, o_vmem):
            pltpu.sync_copy(x_hbm.at[i_vmem.at[0]], o_vmem)

        pltpu.emit_pipeline(
            body,
            grid=(n // _GATHER_WINDOW,),
            in_specs=[pl.BlockSpec((1, _GATHER_WINDOW), lambda i: (0, i))],
            out_specs=[pl.BlockSpec((_GATHER_WINDOW, b), lambda i: (i, 0))],
            core_axis_name=("core", "subcore"),
            dimension_semantics=(pltpu.PARALLEL,),
        )(i_hbm, o_hbm)

    return gather_kernel(sino_t, idx_flat)


def _finish_body(g_ref, w_ref, pi_ref, das_ref):
    e = pl.program_id(0)
    g = g_ref[0]
    gt = jnp.transpose(g, (2, 0, 1))
    wt = gt * w_ref[0][None, :, :]
    pi_ref[:, 0] = wt

    @pl.when(e == 0)
    def _():
        das_ref[:, 0] = wt

    @pl.when(e != 0)
    def _():
        das_ref[:, 0] += wt


def _tc_finish(g4, wm_t, interpret=False):
    e, roi, _, b = g4.shape
    return pl.pallas_call(
        _finish_body,
        grid=(e,),
        in_specs=[
            pl.BlockSpec((1, roi, roi, b), lambda i: (i, 0, 0, 0)),
            pl.BlockSpec((1, roi, roi), lambda i: (i, 0, 0)),
        ],
        out_specs=[
            pl.BlockSpec((b, 1, roi, roi), lambda i: (0, i, 0, 0)),
            pl.BlockSpec((b, 1, roi, roi), lambda i: (0, 0, 0, 0)),
        ],
        out_shape=[
            jax.ShapeDtypeStruct((b, e, roi, roi), jnp.float32),
            jax.ShapeDtypeStruct((b, 1, roi, roi), jnp.float32),
        ],
        interpret=interpret,
    )(g4, wm_t)


def _norm_body(d_ref, o_ref):
    d = jnp.maximum(d_ref[...], 0.0)
    m = jnp.max(d, axis=(1, 2, 3), keepdims=True)
    m = jnp.where(m > 1e-8, m, 1.0)
    o_ref[...] = d / m


def _normalize(das_acc, interpret=False):
    return pl.pallas_call(
        _norm_body,
        out_shape=jax.ShapeDtypeStruct(das_acc.shape, das_acc.dtype),
        interpret=interpret,
    )(das_acc)


def kernel(sinogram, time_indices, weights, valid_mask):
    b, _, e, t = sinogram.shape
    roi = time_indices.shape[0]

    sino_t = sinogram[:, 0].reshape(b, e * t).T
    tc = jnp.clip(time_indices, 0, t - 1).astype(jnp.int32)
    idx = jnp.transpose(tc, (2, 0, 1))
    idx = idx + (jnp.arange(e, dtype=jnp.int32) * t)[:, None, None]
    idx_flat = idx.reshape(1, e * roi * roi)
    wm_t = jnp.transpose(
        jnp.where(valid_mask, weights, 0.0).astype(jnp.float32), (2, 0, 1))

    g = _sc_gather(sino_t.astype(jnp.bfloat16), idx_flat).astype(jnp.float32)
    g4 = g.reshape(e, roi, roi, b)
    pixel_interp, das_acc = _tc_finish(g4, wm_t)
    das = _normalize(das_acc)
    return das, pixel_interp


# speedup vs baseline: 1.4320x; 1.4320x over previous
import functools

import jax
import jax.numpy as jnp
from jax.experimental import pallas as pl
from jax.experimental.pallas import tpu as pltpu
from jax.experimental.pallas import tpu_sc as plsc

_GATHER_WINDOW = 1024


def _sc_gather(sino_t, idx_flat):
    n = idx_flat.shape[1]
    b = sino_t.shape[1]
    mesh = plsc.VectorSubcoreMesh(core_axis_name="core",
                                  subcore_axis_name="subcore")

    @functools.partial(
        pl.kernel,
        out_type=jax.ShapeDtypeStruct((n, b), sino_t.dtype),
        mesh=mesh,
        compiler_params=pltpu.CompilerParams(use_tc_tiling_on_sc=False),
        scratch_types=[pltpu.SemaphoreType.DMA((2,))],
    )
    def gather_kernel(x_hbm, i_hbm, o_hbm, sems):
        half = _GATHER_WINDOW // 2

        def body(i_vmem, o_vmem):
            c0 = pltpu.make_async_copy(
                x_hbm.at[i_vmem.at[0, pl.ds(0, half)]],
                o_vmem.at[pl.ds(0, half)], sems.at[0])
            c1 = pltpu.make_async_copy(
                x_hbm.at[i_vmem.at[0, pl.ds(half, half)]],
                o_vmem.at[pl.ds(half, half)], sems.at[1])
            c0.start()
            c1.start()
            c0.wait()
            c1.wait()

        pltpu.emit_pipeline(
            body,
            grid=(n // _GATHER_WINDOW,),
            in_specs=[pl.BlockSpec((1, _GATHER_WINDOW), lambda i: (0, i))],
            out_specs=[pl.BlockSpec((_GATHER_WINDOW, b), lambda i: (i, 0))],
            core_axis_name=("core", "subcore"),
            dimension_semantics=(pltpu.PARALLEL,),
        )(i_hbm, o_hbm)

    return gather_kernel(sino_t, idx_flat)


def _finish_body(g_ref, w_ref, pi_ref, das_ref):
    e = pl.program_id(0)
    g = g_ref[0]
    gt = jnp.transpose(g, (2, 0, 1))
    wt = gt * w_ref[0][None, :, :]
    pi_ref[:, 0] = wt

    @pl.when(e == 0)
    def _():
        das_ref[:, 0] = wt

    @pl.when(e != 0)
    def _():
        das_ref[:, 0] += wt


def _tc_finish(g4, wm_t, interpret=False):
    e, roi, _, b = g4.shape
    return pl.pallas_call(
        _finish_body,
        grid=(e,),
        in_specs=[
            pl.BlockSpec((1, roi, roi, b), lambda i: (i, 0, 0, 0)),
            pl.BlockSpec((1, roi, roi), lambda i: (i, 0, 0)),
        ],
        out_specs=[
            pl.BlockSpec((b, 1, roi, roi), lambda i: (0, i, 0, 0)),
            pl.BlockSpec((b, 1, roi, roi), lambda i: (0, 0, 0, 0)),
        ],
        out_shape=[
            jax.ShapeDtypeStruct((b, e, roi, roi), jnp.float32),
            jax.ShapeDtypeStruct((b, 1, roi, roi), jnp.float32),
        ],
        interpret=interpret,
    )(g4, wm_t)


def _norm_body(d_ref, o_ref):
    d = jnp.maximum(d_ref[...], 0.0)
    m = jnp.max(d, axis=(1, 2, 3), keepdims=True)
    m = jnp.where(m > 1e-8, m, 1.0)
    o_ref[...] = d / m


def _normalize(das_acc, interpret=False):
    return pl.pallas_call(
        _norm_body,
        out_shape=jax.ShapeDtypeStruct(das_acc.shape, das_acc.dtype),
        interpret=interpret,
    )(das_acc)


def kernel(sinogram, time_indices, weights, valid_mask):
    b, _, e, t = sinogram.shape
    roi = time_indices.shape[0]

    sino_t = sinogram[:, 0].reshape(b, e * t).T
    tc = jnp.clip(time_indices, 0, t - 1).astype(jnp.int32)
    idx = jnp.transpose(tc, (2, 0, 1))
    idx = idx + (jnp.arange(e, dtype=jnp.int32) * t)[:, None, None]
    idx_flat = idx.reshape(1, e * roi * roi)
    wm_t = jnp.transpose(
        jnp.where(valid_mask, weights, 0.0).astype(jnp.float32), (2, 0, 1))

    g = _sc_gather(sino_t, idx_flat)
    g4 = g.reshape(e, roi, roi, b)
    pixel_interp, das_acc = _tc_finish(g4, wm_t)
    das = _normalize(das_acc)
    return das, pixel_interp


# speedup vs baseline: 2.4399x; 1.7039x over previous
import functools

import jax
import jax.numpy as jnp
from jax import lax
from jax.experimental import pallas as pl
from jax.experimental.pallas import tpu as pltpu
from jax.experimental.pallas import tpu_sc as plsc

_CH = 512
_LANES = 16


def _sc_gather(sino3, idx_t):
    b, e, t = sino3.shape
    p = idx_t.shape[1]
    n_tiles = 32
    e_per_tile = e // n_tiles
    n_chunks = p // _CH
    n_groups = _CH // _LANES
    mesh = plsc.VectorSubcoreMesh(core_axis_name="core",
                                  subcore_axis_name="subcore")

    @functools.partial(
        pl.kernel,
        out_type=jax.ShapeDtypeStruct((e, b, p), jnp.float32),
        mesh=mesh,
        compiler_params=pltpu.CompilerParams(use_tc_tiling_on_sc=False,
                                             needs_layout_passes=False),
        scratch_types=[
            pltpu.VMEM((b, t), jnp.float32),
            pltpu.VMEM((p,), jnp.int32),
            pltpu.VMEM((b, _CH), jnp.float32),
            pltpu.VMEM((b, _CH), jnp.float32),
            pltpu.SemaphoreType.DMA,
            pltpu.SemaphoreType.DMA,
            pltpu.SemaphoreType.DMA,
        ],
    )
    def gather_kernel(x_hbm, i_hbm, o_hbm, tbl, idxv, out0, out1, sem_t,
                      sem_w0, sem_w1):
        wid = lax.axis_index("subcore") * 2 + lax.axis_index("core")
        out_slots = (out0, out1)
        sem_slots = (sem_w0, sem_w1)

        def stage(e_idx):
            for bb in range(b):
                pltpu.async_copy(x_hbm.at[bb, e_idx], tbl.at[bb], sem_t)
            pltpu.sync_copy(i_hbm.at[e_idx], idxv)
            for bb in range(b):
                pltpu.make_async_copy(x_hbm.at[bb, e_idx], tbl.at[bb],
                                      sem_t).wait()

        def compute(c, slot):
            out_ref = out_slots[slot]

            @pl.loop(0, n_groups)
            def _(g):
                tvec = idxv[pl.ds(c * _CH + g * _LANES, _LANES)]

                @pl.loop(0, b)
                def _(bb):
                    bvec = jnp.full((_LANES,), bb, jnp.int32)
                    vals = plsc.load_gather(tbl, [bvec, tvec])
                    out_ref[bb, pl.ds(g * _LANES, _LANES)] = vals

        def start_wb(c, slot, e_idx):
            pltpu.async_copy(out_slots[slot],
                             o_hbm.at[e_idx, :, pl.ds(c * _CH, _CH)],
                             sem_slots[slot])

        def wait_wb(slot, e_idx):
            pltpu.make_async_copy(out_slots[slot],
                                  o_hbm.at[e_idx, :, pl.ds(0, _CH)],
                                  sem_slots[slot]).wait()

        for k in range(e_per_tile):
            e_idx = wid * e_per_tile + k
            stage(e_idx)
            if k == 0:
                compute(0, 0)
                start_wb(0, 0, e_idx)
                compute(1, 1)
                start_wb(1, 1, e_idx)

                @pl.loop(2, n_chunks, step=2)
                def _(c):
                    wait_wb(0, e_idx)
                    compute(c, 0)
                    start_wb(c, 0, e_idx)
                    wait_wb(1, e_idx)
                    compute(c + 1, 1)
                    start_wb(c + 1, 1, e_idx)
            else:
                @pl.loop(0, n_chunks, step=2)
                def _(c):
                    wait_wb(0, e_idx)
                    compute(c, 0)
                    start_wb(c, 0, e_idx)
                    wait_wb(1, e_idx)
                    compute(c + 1, 1)
                    start_wb(c + 1, 1, e_idx)
        wait_wb(0, 0)
        wait_wb(1, 0)

    return gather_kernel(sino3, idx_t)


def _finish_body(g_ref, w_ref, pi_ref, das_ref):
    e = pl.program_id(0)
    wt = g_ref[0] * w_ref[0][None]
    pi_ref[:, 0] = wt

    @pl.when(e == 0)
    def _():
        das_ref[:, 0] = wt

    @pl.when(e != 0)
    def _():
        das_ref[:, 0] += wt


def _tc_finish(g4, wm_t, interpret=False):
    e, b, roi, _ = g4.shape
    return pl.pallas_call(
        _finish_body,
        grid=(e,),
        in_specs=[
            pl.BlockSpec((1, b, roi, roi), lambda i: (i, 0, 0, 0)),
            pl.BlockSpec((1, roi, roi), lambda i: (i, 0, 0)),
        ],
        out_specs=[
            pl.BlockSpec((b, 1, roi, roi), lambda i: (0, i, 0, 0)),
            pl.BlockSpec((b, 1, roi, roi), lambda i: (0, 0, 0, 0)),
        ],
        out_shape=[
            jax.ShapeDtypeStruct((b, e, roi, roi), jnp.float32),
            jax.ShapeDtypeStruct((b, 1, roi, roi), jnp.float32),
        ],
        interpret=interpret,
    )(g4, wm_t)


def _norm_body(d_ref, o_ref):
    d = jnp.maximum(d_ref[...], 0.0)
    m = jnp.max(d, axis=(1, 2, 3), keepdims=True)
    m = jnp.where(m > 1e-8, m, 1.0)
    o_ref[...] = d / m


def _normalize(das_acc, interpret=False):
    return pl.pallas_call(
        _norm_body,
        out_shape=jax.ShapeDtypeStruct(das_acc.shape, das_acc.dtype),
        interpret=interpret,
    )(das_acc)


def kernel(sinogram, time_indices, weights, valid_mask):
    b, _, e, t = sinogram.shape
    roi = time_indices.shape[0]
    p = roi * roi

    sino3 = sinogram.reshape(b, e, t)
    tc = jnp.clip(time_indices, 0, t - 1).astype(jnp.int32)
    idx_t = jnp.transpose(tc.reshape(p, e))
    wm_t = jnp.transpose(
        jnp.where(valid_mask, weights, 0.0).astype(jnp.float32), (2, 0, 1))

    g = _sc_gather(sino3, idx_t)
    g4 = g.reshape(e, b, roi, roi)
    pixel_interp, das_acc = _tc_finish(g4, wm_t)
    das = _normalize(das_acc)
    return das, pixel_interp


# speedup vs baseline: 2.6192x; 1.0735x over previous
import functools

import jax
import jax.numpy as jnp
from jax import lax
from jax.experimental import pallas as pl
from jax.experimental.pallas import tpu as pltpu
from jax.experimental.pallas import tpu_sc as plsc

_CH = 512
_LANES = 16


def _sc_gather(sino3, idx_t):
    b, e, t = sino3.shape
    p = idx_t.shape[1]
    n_tiles = 32
    e_per_tile = e // n_tiles
    n_chunks = p // _CH
    n_groups = _CH // _LANES
    mesh = plsc.VectorSubcoreMesh(core_axis_name="core",
                                  subcore_axis_name="subcore")

    @functools.partial(
        pl.kernel,
        out_type=jax.ShapeDtypeStruct((e, b, p), jnp.float32),
        mesh=mesh,
        compiler_params=pltpu.CompilerParams(use_tc_tiling_on_sc=False,
                                             needs_layout_passes=False),
        scratch_types=[
            pltpu.VMEM((b, t), jnp.float32),
            pltpu.VMEM((p,), jnp.int32),
            pltpu.VMEM((b, _CH), jnp.float32),
            pltpu.VMEM((b, _CH), jnp.float32),
            pltpu.SemaphoreType.DMA,
            pltpu.SemaphoreType.DMA,
            pltpu.SemaphoreType.DMA,
        ],
    )
    def gather_kernel(x_hbm, i_hbm, o_hbm, tbl, idxv, out0, out1, sem_t,
                      sem_w0, sem_w1):
        wid = lax.axis_index("subcore") * 2 + lax.axis_index("core")
        out_slots = (out0, out1)
        sem_slots = (sem_w0, sem_w1)

        def stage(e_idx):
            for bb in range(b):
                pltpu.async_copy(x_hbm.at[bb, e_idx], tbl.at[bb], sem_t)
            pltpu.sync_copy(i_hbm.at[e_idx], idxv)
            for bb in range(b):
                pltpu.make_async_copy(x_hbm.at[bb, e_idx], tbl.at[bb],
                                      sem_t).wait()

        def compute(c, slot):
            out_ref = out_slots[slot]

            @pl.loop(0, n_groups)
            def _(g):
                tvec = idxv[pl.ds(c * _CH + g * _LANES, _LANES)]
                for bb in range(b):
                    bvec = jnp.full((_LANES,), bb, jnp.int32)
                    vals = plsc.load_gather(tbl, [bvec, tvec])
                    out_ref[bb, pl.ds(g * _LANES, _LANES)] = vals

        def start_wb(c, slot, e_idx):
            pltpu.async_copy(out_slots[slot],
                             o_hbm.at[e_idx, :, pl.ds(c * _CH, _CH)],
                             sem_slots[slot])

        def wait_wb(slot, e_idx):
            pltpu.make_async_copy(out_slots[slot],
                                  o_hbm.at[e_idx, :, pl.ds(0, _CH)],
                                  sem_slots[slot]).wait()

        for k in range(e_per_tile):
            e_idx = wid * e_per_tile + k
            stage(e_idx)
            if k == 0:
                compute(0, 0)
                start_wb(0, 0, e_idx)
                compute(1, 1)
                start_wb(1, 1, e_idx)

                @pl.loop(2, n_chunks, step=2)
                def _(c):
                    wait_wb(0, e_idx)
                    compute(c, 0)
                    start_wb(c, 0, e_idx)
                    wait_wb(1, e_idx)
                    compute(c + 1, 1)
                    start_wb(c + 1, 1, e_idx)
            else:
                @pl.loop(0, n_chunks, step=2)
                def _(c):
                    wait_wb(0, e_idx)
                    compute(c, 0)
                    start_wb(c, 0, e_idx)
                    wait_wb(1, e_idx)
                    compute(c + 1, 1)
                    start_wb(c + 1, 1, e_idx)
        wait_wb(0, 0)
        wait_wb(1, 0)

    return gather_kernel(sino3, idx_t)


def _finish_body(g_ref, w_ref, pi_ref, das_ref):
    e = pl.program_id(0)
    wt = g_ref[0] * w_ref[0][None]
    pi_ref[:, 0] = wt

    @pl.when(e == 0)
    def _():
        das_ref[:, 0] = wt

    @pl.when(e != 0)
    def _():
        das_ref[:, 0] += wt


def _tc_finish(g4, wm_t, interpret=False):
    e, b, roi, _ = g4.shape
    return pl.pallas_call(
        _finish_body,
        grid=(e,),
        in_specs=[
            pl.BlockSpec((1, b, roi, roi), lambda i: (i, 0, 0, 0)),
            pl.BlockSpec((1, roi, roi), lambda i: (i, 0, 0)),
        ],
        out_specs=[
            pl.BlockSpec((b, 1, roi, roi), lambda i: (0, i, 0, 0)),
            pl.BlockSpec((b, 1, roi, roi), lambda i: (0, 0, 0, 0)),
        ],
        out_shape=[
            jax.ShapeDtypeStruct((b, e, roi, roi), jnp.float32),
            jax.ShapeDtypeStruct((b, 1, roi, roi), jnp.float32),
        ],
        interpret=interpret,
    )(g4, wm_t)


def _norm_body(d_ref, o_ref):
    d = jnp.maximum(d_ref[...], 0.0)
    m = jnp.max(d, axis=(1, 2, 3), keepdims=True)
    m = jnp.where(m > 1e-8, m, 1.0)
    o_ref[...] = d / m


def _normalize(das_acc, interpret=False):
    return pl.pallas_call(
        _norm_body,
        out_shape=jax.ShapeDtypeStruct(das_acc.shape, das_acc.dtype),
        interpret=interpret,
    )(das_acc)


def kernel(sinogram, time_indices, weights, valid_mask):
    b, _, e, t = sinogram.shape
    roi = time_indices.shape[0]
    p = roi * roi

    sino3 = sinogram.reshape(b, e, t)
    tc = jnp.clip(time_indices, 0, t - 1).astype(jnp.int32)
    idx_t = jnp.transpose(tc.reshape(p, e))
    wm_t = jnp.transpose(
        jnp.where(valid_mask, weights, 0.0).astype(jnp.float32), (2, 0, 1))

    g = _sc_gather(sino3, idx_t)
    g4 = g.reshape(e, b, roi, roi)
    pixel_interp, das_acc = _tc_finish(g4, wm_t)
    das = _normalize(das_acc)
    return das, pixel_interp


# speedup vs baseline: 2.6987x; 1.0304x over previous
import functools

import jax
import jax.numpy as jnp
from jax import lax
from jax.experimental import pallas as pl
from jax.experimental.pallas import tpu as pltpu
from jax.experimental.pallas import tpu_sc as plsc

_CH = 512
_LANES = 16


def _sc_gather(sino3, idx_t):
    b, e, t = sino3.shape
    p = idx_t.shape[1]
    n_tiles = 32
    e_per_tile = e // n_tiles
    n_chunks = p // _CH
    n_groups = _CH // _LANES
    mesh = plsc.VectorSubcoreMesh(core_axis_name="core",
                                  subcore_axis_name="subcore")

    @functools.partial(
        pl.kernel,
        out_type=jax.ShapeDtypeStruct((e, b, p), jnp.float32),
        mesh=mesh,
        compiler_params=pltpu.CompilerParams(use_tc_tiling_on_sc=False,
                                             needs_layout_passes=False),
        scratch_types=[
            pltpu.VMEM((b, t), jnp.float32),
            pltpu.VMEM((p,), jnp.int32),
            pltpu.VMEM((b, _CH), jnp.float32),
            pltpu.VMEM((b, _CH), jnp.float32),
            pltpu.SemaphoreType.DMA,
            pltpu.SemaphoreType.DMA,
            pltpu.SemaphoreType.DMA,
        ],
    )
    def gather_kernel(x_hbm, i_hbm, o_hbm, tbl, idxv, out0, out1, sem_t,
                      sem_w0, sem_w1):
        wid = lax.axis_index("subcore") * 2 + lax.axis_index("core")
        out_slots = (out0, out1)
        sem_slots = (sem_w0, sem_w1)

        def stage(e_idx):
            for bb in range(b):
                pltpu.async_copy(x_hbm.at[bb, e_idx], tbl.at[bb], sem_t)
            pltpu.sync_copy(i_hbm.at[e_idx], idxv)
            for bb in range(b):
                pltpu.make_async_copy(x_hbm.at[bb, e_idx], tbl.at[bb],
                                      sem_t).wait()

        def compute(c, slot):
            out_ref = out_slots[slot]

            @pl.loop(0, n_groups, step=2)
            def _(g):
                for gg in range(2):
                    tvec = idxv[pl.ds(c * _CH + (g + gg) * _LANES, _LANES)]
                    for bb in range(b):
                        bvec = jnp.full((_LANES,), bb, jnp.int32)
                        vals = plsc.load_gather(tbl, [bvec, tvec])
                        out_ref[bb, pl.ds((g + gg) * _LANES, _LANES)] = vals

        def start_wb(c, slot, e_idx):
            pltpu.async_copy(out_slots[slot],
                             o_hbm.at[e_idx, :, pl.ds(c * _CH, _CH)],
                             sem_slots[slot])

        def wait_wb(slot, e_idx):
            pltpu.make_async_copy(out_slots[slot],
                                  o_hbm.at[e_idx, :, pl.ds(0, _CH)],
                                  sem_slots[slot]).wait()

        for k in range(e_per_tile):
            e_idx = wid * e_per_tile + k
            stage(e_idx)
            if k == 0:
                compute(0, 0)
                start_wb(0, 0, e_idx)
                compute(1, 1)
                start_wb(1, 1, e_idx)

                @pl.loop(2, n_chunks, step=2)
                def _(c):
                    wait_wb(0, e_idx)
                    compute(c, 0)
                    start_wb(c, 0, e_idx)
                    wait_wb(1, e_idx)
                    compute(c + 1, 1)
                    start_wb(c + 1, 1, e_idx)
            else:
                @pl.loop(0, n_chunks, step=2)
                def _(c):
                    wait_wb(0, e_idx)
                    compute(c, 0)
                    start_wb(c, 0, e_idx)
                    wait_wb(1, e_idx)
                    compute(c + 1, 1)
                    start_wb(c + 1, 1, e_idx)
        wait_wb(0, 0)
        wait_wb(1, 0)

    return gather_kernel(sino3, idx_t)


def _finish_body(g_ref, w_ref, pi_ref, das_ref):
    e = pl.program_id(0)
    wt = g_ref[0] * w_ref[0][None]
    pi_ref[:, 0] = wt

    @pl.when(e == 0)
    def _():
        das_ref[:, 0] = wt

    @pl.when(e != 0)
    def _():
        das_ref[:, 0] += wt


def _tc_finish(g4, wm_t, interpret=False):
    e, b, roi, _ = g4.shape
    return pl.pallas_call(
        _finish_body,
        grid=(e,),
        in_specs=[
            pl.BlockSpec((1, b, roi, roi), lambda i: (i, 0, 0, 0)),
            pl.BlockSpec((1, roi, roi), lambda i: (i, 0, 0)),
        ],
        out_specs=[
            pl.BlockSpec((b, 1, roi, roi), lambda i: (0, i, 0, 0)),
            pl.BlockSpec((b, 1, roi, roi), lambda i: (0, 0, 0, 0)),
        ],
        out_shape=[
            jax.ShapeDtypeStruct((b, e, roi, roi), jnp.float32),
            jax.ShapeDtypeStruct((b, 1, roi, roi), jnp.float32),
        ],
        interpret=interpret,
    )(g4, wm_t)


def _norm_body(d_ref, o_ref):
    d = jnp.maximum(d_ref[...], 0.0)
    m = jnp.max(d, axis=(1, 2, 3), keepdims=True)
    m = jnp.where(m > 1e-8, m, 1.0)
    o_ref[...] = d / m


def _normalize(das_acc, interpret=False):
    return pl.pallas_call(
        _norm_body,
        out_shape=jax.ShapeDtypeStruct(das_acc.shape, das_acc.dtype),
        interpret=interpret,
    )(das_acc)


def kernel(sinogram, time_indices, weights, valid_mask):
    b, _, e, t = sinogram.shape
    roi = time_indices.shape[0]
    p = roi * roi

    sino3 = sinogram.reshape(b, e, t)
    tc = jnp.clip(time_indices, 0, t - 1).astype(jnp.int32)
    idx_t = jnp.transpose(tc.reshape(p, e))
    wm_t = jnp.transpose(
        jnp.where(valid_mask, weights, 0.0).astype(jnp.float32), (2, 0, 1))

    g = _sc_gather(sino3, idx_t)
    g4 = g.reshape(e, b, roi, roi)
    pixel_interp, das_acc = _tc_finish(g4, wm_t)
    das = _normalize(das_acc)
    return das, pixel_interp


# speedup vs baseline: 2.9446x; 1.0911x over previous
import functools

import jax
import jax.numpy as jnp
from jax import lax
from jax.experimental import pallas as pl
from jax.experimental.pallas import tpu as pltpu
from jax.experimental.pallas import tpu_sc as plsc

_CH = 512
_LANES = 16
_N_TILES = 32


def _sc_gather(sino3, idx_t, e_base, ne):
    b, _, t = sino3.shape
    p = idx_t.shape[1]
    e_per_tile = ne // _N_TILES
    n_chunks = p // _CH
    n_groups = _CH // _LANES
    mesh = plsc.VectorSubcoreMesh(core_axis_name="core",
                                  subcore_axis_name="subcore")

    @functools.partial(
        pl.kernel,
        out_type=jax.ShapeDtypeStruct((ne, b, p), jnp.float32),
        mesh=mesh,
        compiler_params=pltpu.CompilerParams(use_tc_tiling_on_sc=False,
                                             needs_layout_passes=False),
        scratch_types=[
            pltpu.VMEM((b, t), jnp.float32),
            pltpu.VMEM((p,), jnp.int32),
            pltpu.VMEM((b, _CH), jnp.float32),
            pltpu.VMEM((b, _CH), jnp.float32),
            pltpu.SemaphoreType.DMA,
            pltpu.SemaphoreType.DMA,
            pltpu.SemaphoreType.DMA,
        ],
    )
    def gather_kernel(x_hbm, i_hbm, o_hbm, tbl, idxv, out0, out1, sem_t,
                      sem_w0, sem_w1):
        wid = lax.axis_index("subcore") * 2 + lax.axis_index("core")
        out_slots = (out0, out1)
        sem_slots = (sem_w0, sem_w1)

        def stage(e_idx):
            for bb in range(b):
                pltpu.async_copy(x_hbm.at[bb, e_idx], tbl.at[bb], sem_t)
            pltpu.sync_copy(i_hbm.at[e_idx], idxv)
            for bb in range(b):
                pltpu.make_async_copy(x_hbm.at[bb, e_idx], tbl.at[bb],
                                      sem_t).wait()

        def compute(c, slot):
            out_ref = out_slots[slot]

            @pl.loop(0, n_groups, step=2)
            def _(g):
                for gg in range(2):
                    tvec = idxv[pl.ds(c * _CH + (g + gg) * _LANES, _LANES)]
                    for bb in range(b):
                        bvec = jnp.full((_LANES,), bb, jnp.int32)
                        vals = plsc.load_gather(tbl, [bvec, tvec])
                        out_ref[bb, pl.ds((g + gg) * _LANES, _LANES)] = vals

        def start_wb(c, slot, o_idx):
            pltpu.async_copy(out_slots[slot],
                             o_hbm.at[o_idx, :, pl.ds(c * _CH, _CH)],
                             sem_slots[slot])

        def wait_wb(slot):
            pltpu.make_async_copy(out_slots[slot],
                                  o_hbm.at[0, :, pl.ds(0, _CH)],
                                  sem_slots[slot]).wait()

        for k in range(e_per_tile):
            o_idx = wid * e_per_tile + k
            e_idx = e_base + o_idx
            stage(e_idx)
            if k == 0:
                compute(0, 0)
                start_wb(0, 0, o_idx)
                compute(1, 1)
                start_wb(1, 1, o_idx)

                @pl.loop(2, n_chunks, step=2)
                def _(c):
                    wait_wb(0)
                    compute(c, 0)
                    start_wb(c, 0, o_idx)
                    wait_wb(1)
                    compute(c + 1, 1)
                    start_wb(c + 1, 1, o_idx)
            else:
                @pl.loop(0, n_chunks, step=2)
                def _(c):
                    wait_wb(0)
                    compute(c, 0)
                    start_wb(c, 0, o_idx)
                    wait_wb(1)
                    compute(c + 1, 1)
                    start_wb(c + 1, 1, o_idx)
        wait_wb(0)
        wait_wb(1)

    return gather_kernel(sino3, idx_t)


def _finish_body(g_ref, w_ref, *rest):
    *maybe_prev, pi_ref, das_ref = rest
    del maybe_prev
    e = pl.program_id(0)
    wt = g_ref[0] * w_ref[0][None]
    pi_ref[:, 0] = wt

    @pl.when(e == 0)
    def _():
        das_ref[:, 0] = wt

    @pl.when(e != 0)
    def _():
        das_ref[:, 0] += wt


def _tc_finish(g_h, wm_t, e_base, e_total, pi_prev=None, interpret=False):
    ne, b, roi, _ = g_h.shape
    in_specs = [
        pl.BlockSpec((1, b, roi, roi), lambda i: (i, 0, 0, 0)),
        pl.BlockSpec((1, roi, roi), lambda i, eb=e_base: (eb + i, 0, 0)),
    ]
    args = (g_h, wm_t)
    aliases = {}
    if pi_prev is not None:
        in_specs.append(pl.BlockSpec(memory_space=pl.ANY))
        args = (g_h, wm_t, pi_prev)
        aliases = {2: 0}
    return pl.pallas_call(
        _finish_body,
        grid=(ne,),
        in_specs=in_specs,
        out_specs=[
            pl.BlockSpec((b, 1, roi, roi),
                         lambda i, eb=e_base: (0, eb + i, 0, 0)),
            pl.BlockSpec((b, 1, roi, roi), lambda i: (0, 0, 0, 0)),
        ],
        out_shape=[
            jax.ShapeDtypeStruct((b, e_total, roi, roi), jnp.float32),
            jax.ShapeDtypeStruct((b, 1, roi, roi), jnp.float32),
        ],
        input_output_aliases=aliases,
        interpret=interpret,
    )(*args)


def _norm_body(d1_ref, d2_ref, o_ref):
    d = jnp.maximum(d1_ref[...] + d2_ref[...], 0.0)
    m = jnp.max(d, axis=(1, 2, 3), keepdims=True)
    m = jnp.where(m > 1e-8, m, 1.0)
    o_ref[...] = d / m


def _normalize(das1, das2, interpret=False):
    return pl.pallas_call(
        _norm_body,
        out_shape=jax.ShapeDtypeStruct(das1.shape, das1.dtype),
        interpret=interpret,
    )(das1, das2)


def kernel(sinogram, time_indices, weights, valid_mask):
    b, _, e, t = sinogram.shape
    roi = time_indices.shape[0]
    p = roi * roi
    ne = e // 2

    sino3 = sinogram.reshape(b, e, t)
    tc = jnp.clip(time_indices, 0, t - 1).astype(jnp.int32)
    idx_t = jnp.transpose(tc.reshape(p, e))
    wm_t = jnp.transpose(
        jnp.where(valid_mask, weights, 0.0).astype(jnp.float32), (2, 0, 1))

    g0 = _sc_gather(sino3, idx_t, 0, ne)
    g1 = _sc_gather(sino3, idx_t, ne, ne)
    pi1, das0 = _tc_finish(g0.reshape(ne, b, roi, roi), wm_t, 0, e)
    pixel_interp, das1 = _tc_finish(g1.reshape(ne, b, roi, roi), wm_t, ne,
                                    e, pi1)
    das = _normalize(das0, das1)
    return das, pixel_interp


# speedup vs baseline: 2.9477x; 1.0010x over previous
import functools

import jax
import jax.numpy as jnp
from jax import lax
from jax.experimental import pallas as pl
from jax.experimental.pallas import tpu as pltpu
from jax.experimental.pallas import tpu_sc as plsc

_CH = 512
_LANES = 16
_N_TILES = 32
_TWIN = 1024


def _sc_gather(sino3, idx_t, e_base, ne):
    b, e_tot, t = sino3.shape
    p = idx_t.shape[1]
    e_per_tile = ne // _N_TILES
    n_chunks = p // _CH
    n_groups = _CH // _LANES
    mesh = plsc.VectorSubcoreMesh(core_axis_name="core",
                                  subcore_axis_name="subcore")

    @functools.partial(
        pl.kernel,
        out_type=jax.ShapeDtypeStruct((ne, b, p), jnp.float32),
        mesh=mesh,
        compiler_params=pltpu.CompilerParams(use_tc_tiling_on_sc=False,
                                             needs_layout_passes=False),
        scratch_types=[
            pltpu.VMEM((b, t), jnp.float32),
            pltpu.VMEM((p,), jnp.int32),
            pltpu.VMEM((b, _CH), jnp.float32),
            pltpu.VMEM((b, _CH), jnp.float32),
            pltpu.SemaphoreType.DMA,
            pltpu.SemaphoreType.DMA,
            pltpu.SemaphoreType.DMA,
        ],
    )
    def gather_kernel(x_hbm, i_hbm, o_hbm, tbl, idxv, out0, out1,
                      sem_t, sem_w0, sem_w1):
        wid = lax.axis_index("subcore") * 2 + lax.axis_index("core")
        out_slots = (out0, out1)
        sem_slots = (sem_w0, sem_w1)

        def stage(e_idx):
            for bb in range(b):
                pltpu.async_copy(x_hbm.at[bb, e_idx], tbl.at[bb], sem_t)
            pltpu.sync_copy(i_hbm.at[e_idx], idxv)
            for bb in range(b):
                pltpu.make_async_copy(x_hbm.at[bb, e_idx], tbl.at[bb],
                                      sem_t).wait()

        def compute(c, slot):
            out_ref = out_slots[slot]

            @pl.loop(0, n_groups, step=2)
            def _(g):
                for gg in range(2):
                    tvec = idxv[pl.ds(c * _CH + (g + gg) * _LANES, _LANES)]
                    for bb in range(b):
                        bvec = jnp.full((_LANES,), bb, jnp.int32)
                        vals = plsc.load_gather(tbl, [bvec, tvec])
                        out_ref[bb, pl.ds((g + gg) * _LANES, _LANES)] = vals

        def start_wb(c, slot, o_idx):
            pltpu.async_copy(out_slots[slot],
                             o_hbm.at[o_idx, :, pl.ds(c * _CH, _CH)],
                             sem_slots[slot])

        def wait_wb(slot):
            pltpu.make_async_copy(out_slots[slot],
                                  o_hbm.at[0, :, pl.ds(0, _CH)],
                                  sem_slots[slot]).wait()

        for k in range(e_per_tile):
            o_idx = wid * e_per_tile + k
            e_idx = e_base + o_idx
            stage(e_idx)
            if k == 0:
                compute(0, 0)
                start_wb(0, 0, o_idx)
                compute(1, 1)
                start_wb(1, 1, o_idx)

                @pl.loop(2, n_chunks, step=2)
                def _(c):
                    wait_wb(0)
                    compute(c, 0)
                    start_wb(c, 0, o_idx)
                    wait_wb(1)
                    compute(c + 1, 1)
                    start_wb(c + 1, 1, o_idx)
            else:
                @pl.loop(0, n_chunks, step=2)
                def _(c):
                    wait_wb(0)
                    compute(c, 0)
                    start_wb(c, 0, o_idx)
                    wait_wb(1)
                    compute(c + 1, 1)
                    start_wb(c + 1, 1, o_idx)
        wait_wb(0)
        wait_wb(1)

    return gather_kernel(sino3, idx_t)


def _finish_body(g_ref, w_ref, *rest):
    *maybe_prev, pi_ref, das_ref = rest
    del maybe_prev
    e = pl.program_id(0)
    wt = g_ref[0] * w_ref[0][None]
    pi_ref[:, 0] = wt

    @pl.when(e == 0)
    def _():
        das_ref[:, 0] = wt

    @pl.when(e != 0)
    def _():
        das_ref[:, 0] += wt


def _tc_finish(g_h, wm_t, e_base, e_total, pi_prev=None, interpret=False):
    ne, b, roi, _ = g_h.shape
    in_specs = [
        pl.BlockSpec((1, b, roi, roi), lambda i: (i, 0, 0, 0)),
        pl.BlockSpec((1, roi, roi), lambda i, eb=e_base: (eb + i, 0, 0)),
    ]
    args = (g_h, wm_t)
    aliases = {}
    if pi_prev is not None:
        in_specs.append(pl.BlockSpec(memory_space=pl.ANY))
        args = (g_h, wm_t, pi_prev)
        aliases = {2: 0}
    return pl.pallas_call(
        _finish_body,
        grid=(ne,),
        in_specs=in_specs,
        out_specs=[
            pl.BlockSpec((b, 1, roi, roi),
                         lambda i, eb=e_base: (0, eb + i, 0, 0)),
            pl.BlockSpec((b, 1, roi, roi), lambda i: (0, 0, 0, 0)),
        ],
        out_shape=[
            jax.ShapeDtypeStruct((b, e_total, roi, roi), jnp.float32),
            jax.ShapeDtypeStruct((b, 1, roi, roi), jnp.float32),
        ],
        input_output_aliases=aliases,
        interpret=interpret,
    )(*args)


def _norm_body(d1_ref, d2_ref, o_ref):
    d = jnp.maximum(d1_ref[...] + d2_ref[...], 0.0)
    m = jnp.max(d, axis=(1, 2, 3), keepdims=True)
    m = jnp.where(m > 1e-8, m, 1.0)
    o_ref[...] = d / m


def _normalize(das1, das2, interpret=False):
    return pl.pallas_call(
        _norm_body,
        out_shape=jax.ShapeDtypeStruct(das1.shape, das1.dtype),
        interpret=interpret,
    )(das1, das2)


def kernel(sinogram, time_indices, weights, valid_mask):
    b, _, e, t = sinogram.shape
    roi = time_indices.shape[0]
    p = roi * roi
    ne = e // 2

    sino3 = sinogram.reshape(b, e, t)
    tc = jnp.clip(time_indices, 0, t - 1).astype(jnp.int32)
    idx_t = jnp.transpose(tc.reshape(p, e))
    wm_t = jnp.transpose(
        jnp.where(valid_mask, weights, 0.0).astype(jnp.float32), (2, 0, 1))

    g0 = _sc_gather(sino3, idx_t, 0, ne)
    g1 = _sc_gather(sino3, idx_t, ne, ne)
    pi1, das0 = _tc_finish(g0.reshape(ne, b, roi, roi), wm_t, 0, e)
    pixel_interp, das1 = _tc_finish(g1.reshape(ne, b, roi, roi), wm_t, ne,
                                    e, pi1)
    das = _normalize(das0, das1)
    return das, pixel_interp
